# Initial kernel scaffold; baseline (speedup 1.0000x reference)
import jax
import jax.numpy as jnp
from jax.experimental import pallas as pl


def kernel(x, edge_index, W_l, W_r, b):
    raise NotImplementedError("write your pallas kernel here")


# speedup vs baseline: 5.7138x; 5.7138x over previous
import functools

import jax
import jax.numpy as jnp
from jax import lax
from jax.experimental import pallas as pl
from jax.experimental.pallas import tpu as pltpu
from jax.experimental.pallas import tpu_sc as plsc

N_SRC = 10000
N_DST = 2048
E = 320000
D = 128

NC = 2
NS = 16
NW = NC * NS
EDGES_PER_TILE = E // NW
CHUNK = 80
NCHUNK = EDGES_PER_TILE // CHUNK
ROWS_PER_TILE = N_DST // NS


CNT_W = 128


def _sc_aggregate(x, src, dst, z_acc):
    mesh = plsc.VectorSubcoreMesh(core_axis_name="c", subcore_axis_name="s")

    @functools.partial(
        pl.kernel,
        mesh=mesh,
        out_type=(
            jax.ShapeDtypeStruct((NC, N_DST, D), jnp.float32),
            jax.ShapeDtypeStruct((NC, N_DST, CNT_W), jnp.float32),
        ),
        scratch_types=[
            pltpu.VMEM((CHUNK,), jnp.int32),
            pltpu.VMEM((CHUNK,), jnp.int32),
            pltpu.VMEM((CHUNK, D), jnp.float32),
            pltpu.VMEM((CHUNK, CNT_W), jnp.float32),
            pltpu.VMEM_SHARED((N_DST, D), jnp.float32),
            pltpu.VMEM_SHARED((N_DST, CNT_W), jnp.float32),
            pltpu.SemaphoreType.DMA,
        ],
    )
    def k(x_hbm, src_hbm, dst_hbm, zacc_hbm, acc_out, cnt_out,
          src_v, dst_v, rows_v, ones_v, acc_sh, cnt_sh, sem):
        cid = lax.axis_index("c")
        sid = lax.axis_index("s")
        wid = cid * NS + sid
        base = wid * EDGES_PER_TILE
        row0 = sid * ROWS_PER_TILE

        pltpu.sync_copy(zacc_hbm.at[pl.ds(row0, ROWS_PER_TILE)],
                        acc_sh.at[pl.ds(row0, ROWS_PER_TILE)])
        one16 = jnp.full((16,), 1.0, dtype=jnp.float32)
        for r in range(CHUNK):
            for j in range(CNT_W // 16):
                ones_v[r, pl.ds(j * 16, 16)] = one16
        pltpu.sync_copy(zacc_hbm.at[pl.ds(row0, ROWS_PER_TILE)],
                        cnt_sh.at[pl.ds(row0, ROWS_PER_TILE)])

        plsc.subcore_barrier()

        def body(i, carry):
            off = base + i * CHUNK
            pltpu.sync_copy(src_hbm.at[pl.ds(off, CHUNK)], src_v)
            pltpu.sync_copy(dst_hbm.at[pl.ds(off, CHUNK)], dst_v)
            pltpu.async_copy(x_hbm.at[src_v], rows_v, sem).wait()
            pltpu.sync_copy(rows_v, acc_sh.at[dst_v], add=True)
            pltpu.sync_copy(ones_v, cnt_sh.at[dst_v], add=True)
            return carry

        lax.fori_loop(0, NCHUNK, body, 0)

        plsc.subcore_barrier()

        pltpu.sync_copy(acc_sh.at[pl.ds(row0, ROWS_PER_TILE)],
                        acc_out.at[cid, pl.ds(row0, ROWS_PER_TILE)])
        pltpu.sync_copy(cnt_sh.at[pl.ds(row0, ROWS_PER_TILE)],
                        cnt_out.at[cid, pl.ds(row0, ROWS_PER_TILE)])

    return k(x, src, dst, z_acc)


def _tc_body(acc_ref, cnt_ref, xt_ref, wlt_ref, wrt_ref, b_ref, o_ref):
    acc = acc_ref[0] + acc_ref[1]
    cnt = (cnt_ref[0] + cnt_ref[1])[:, None]
    agg = acc / jnp.maximum(cnt, 1.0)
    out = (jnp.dot(agg, wlt_ref[...], preferred_element_type=jnp.float32)
           + b_ref[...]
           + jnp.dot(xt_ref[...], wrt_ref[...],
                     preferred_element_type=jnp.float32))
    norm = jnp.sqrt(jnp.sum(out * out, axis=1, keepdims=True))
    o_ref[...] = out / jnp.maximum(norm, 1e-12)


def kernel(x, edge_index, W_l, W_r, b):
    src = edge_index[0].astype(jnp.int32)
    dst = edge_index[1].astype(jnp.int32)
    z_acc = jnp.zeros((N_DST, D), dtype=jnp.float32)

    acc_p, cnt_p = _sc_aggregate(x, src, dst, z_acc)

    out = pl.pallas_call(
        _tc_body,
        out_shape=jax.ShapeDtypeStruct((N_DST, D), jnp.float32),
    )(acc_p, cnt_p[:, :, 0], x[:N_DST], W_l.T, W_r.T, b.reshape(1, D))
    return out
